# Initial kernel scaffold; baseline (speedup 1.0000x reference)
import jax
import jax.numpy as jnp
from jax.experimental import pallas as pl


def kernel(x, edge_index, W1, b1, W2, b2):
    raise NotImplementedError("write your pallas kernel here")


# speedup vs baseline: 24.3123x; 24.3123x over previous
import functools

import jax
import jax.numpy as jnp
from jax import lax
from jax.experimental import pallas as pl
from jax.experimental.pallas import tpu as pltpu
from jax.experimental.pallas import tpu_sc as plsc

N = 10000
E = 320000
NCORES = 2
NSUB = 16
STRIPE = 632
CHUNK = 80

DEG_EDGES_PER_W = E // (NCORES * NSUB)
DEG_CHUNKS = DEG_EDGES_PER_W // CHUNK
AGG_EDGES_PER_T = E // NSUB
AGG_CHUNKS = AGG_EDGES_PER_T // CHUNK

_MESH = plsc.VectorSubcoreMesh(core_axis_name="c", subcore_axis_name="s")
_SC_PARAMS = pltpu.CompilerParams(use_tc_tiling_on_sc=False)


def _deg_body(dst_hbm, ones_hbm, out_hbm, idx_v, ones_v, acc_sh, sem):
    c = lax.axis_index("c")
    s = lax.axis_index("s")
    wid = s * NCORES + c
    base = pl.multiple_of(jnp.minimum(s * STRIPE, N - STRIPE), 8)
    pltpu.sync_copy(ones_hbm.at[pl.ds(base, STRIPE)],
                    acc_sh.at[pl.ds(base, STRIPE)])
    pltpu.sync_copy(dst_hbm.at[wid], idx_v)
    pltpu.sync_copy(ones_hbm.at[pl.ds(0, CHUNK)], ones_v)
    plsc.subcore_barrier()

    def body(j, carry):
        pltpu.sync_copy(ones_v, acc_sh.at[idx_v.at[j]], add=True)
        return carry

    lax.fori_loop(0, DEG_CHUNKS, body, 0)
    plsc.subcore_barrier()
    pltpu.sync_copy(acc_sh.at[pl.ds(base, STRIPE)],
                    out_hbm.at[c, pl.ds(base, STRIPE)])


_deg_kernel = pl.kernel(
    _deg_body,
    out_type=jax.ShapeDtypeStruct((NCORES, N, 16), jnp.float32),
    mesh=_MESH,
    compiler_params=_SC_PARAMS,
    scratch_types=[
        pltpu.VMEM((DEG_CHUNKS, CHUNK), jnp.int32),
        pltpu.VMEM((CHUNK, 16), jnp.float32),
        pltpu.VMEM_SHARED((N, 16), jnp.float32),
        pltpu.SemaphoreType.DMA,
    ],
)


def _agg_body(hs_hbm, src_hbm, dst_hbm, out_hbm,
              src_v, dst_v, rows0, rows1, acc_sh, sem0, sem1, *, dh):
    c = lax.axis_index("c")
    s = lax.axis_index("s")
    base = pl.multiple_of(jnp.minimum(s * STRIPE, N - STRIPE), 8)
    pltpu.sync_copy(hs_hbm.at[c, pl.ds(base, STRIPE)],
                    acc_sh.at[pl.ds(base, STRIPE)])
    pltpu.sync_copy(src_hbm.at[s], src_v)
    pltpu.sync_copy(dst_hbm.at[s], dst_v)
    plsc.subcore_barrier()

    pltpu.async_copy(hs_hbm.at[c].at[src_v.at[0]], rows0, sem0)
    pltpu.async_copy(hs_hbm.at[c].at[src_v.at[1]], rows1, sem1)

    def waitg(buf, sem):
        pltpu.make_async_copy(hs_hbm.at[c, pl.ds(0, CHUNK)], buf, sem).wait()

    def body(j2, carry):
        j = 2 * j2
        waitg(rows0, sem0)
        pltpu.sync_copy(rows0, acc_sh.at[dst_v.at[j]], add=True)

        @pl.when(j + 2 < AGG_CHUNKS)
        def _():
            pltpu.async_copy(hs_hbm.at[c].at[src_v.at[j + 2]], rows0, sem0)

        waitg(rows1, sem1)
        pltpu.sync_copy(rows1, acc_sh.at[dst_v.at[j + 1]], add=True)

        @pl.when(j + 3 < AGG_CHUNKS)
        def _():
            pltpu.async_copy(hs_hbm.at[c].at[src_v.at[j + 3]], rows1, sem1)

        return carry

    lax.fori_loop(0, AGG_CHUNKS // 2, body, 0)
    plsc.subcore_barrier()
    pltpu.sync_copy(acc_sh.at[pl.ds(base, STRIPE)],
                    out_hbm.at[c, pl.ds(base, STRIPE)])


def _make_agg_kernel(dh):
    return pl.kernel(
        functools.partial(_agg_body, dh=dh),
        out_type=jax.ShapeDtypeStruct((NCORES, N, dh), jnp.float32),
        mesh=_MESH,
        compiler_params=_SC_PARAMS,
        scratch_types=[
            pltpu.VMEM((AGG_CHUNKS, CHUNK), jnp.int32),
            pltpu.VMEM((AGG_CHUNKS, CHUNK), jnp.int32),
            pltpu.VMEM((CHUNK, dh), jnp.float32),
            pltpu.VMEM((CHUNK, dh), jnp.float32),
            pltpu.VMEM_SHARED((N, dh), jnp.float32),
            pltpu.SemaphoreType.DMA,
            pltpu.SemaphoreType.DMA,
        ],
    )


_agg64 = _make_agg_kernel(64)
_agg32 = _make_agg_kernel(32)


ROW_BLK = 1000


def _dinv_block(dpart_ref):
    deg = dpart_ref[0, :, 0:1] + dpart_ref[1, :, 0:1] - 1.0
    return lax.rsqrt(deg)


def _tc1_body(dpart_ref, x_ref, w1_ref, out_ref):
    dinv = _dinv_block(dpart_ref)
    h = jnp.dot(x_ref[...], w1_ref[...], preferred_element_type=jnp.float32)
    hs = h * dinv
    out_ref[0, :, :] = hs[:, :64]
    out_ref[1, :, :] = hs[:, 64:]


def _tc2_body(dpart_ref, acc_ref, b1_ref, w2_ref, out_ref):
    dinv = _dinv_block(dpart_ref)
    z0 = jnp.maximum(acc_ref[0, :, :] * dinv + b1_ref[0, :64], 0.0)
    z1 = jnp.maximum(acc_ref[1, :, :] * dinv + b1_ref[0, 64:], 0.0)
    h = (jnp.dot(z0, w2_ref[:64, :], preferred_element_type=jnp.float32)
         + jnp.dot(z1, w2_ref[64:, :], preferred_element_type=jnp.float32))
    hs = h * dinv
    out_ref[0, :, :] = hs[:, :32]
    out_ref[1, :, :] = hs[:, 32:]


def _tc3_body(dpart_ref, acc_ref, b2_ref, out_ref):
    dinv = _dinv_block(dpart_ref)
    lo = acc_ref[0, :, :] * dinv + b2_ref[0, :32]
    hi = acc_ref[1, :, :] * dinv + b2_ref[0, 32:]
    out_ref[...] = jnp.concatenate([lo, hi], axis=1)


_N_BLKS = N // ROW_BLK

_dpart_spec = pl.BlockSpec((NCORES, ROW_BLK, 16), lambda i: (0, i, 0))

_tc1 = pl.pallas_call(
    _tc1_body,
    grid=(_N_BLKS,),
    in_specs=[
        _dpart_spec,
        pl.BlockSpec((ROW_BLK, 128), lambda i: (i, 0)),
        pl.BlockSpec((128, 128), lambda i: (0, 0)),
    ],
    out_specs=pl.BlockSpec((NCORES, ROW_BLK, 64), lambda i: (0, i, 0)),
    out_shape=jax.ShapeDtypeStruct((NCORES, N, 64), jnp.float32),
)

_tc2 = pl.pallas_call(
    _tc2_body,
    grid=(_N_BLKS,),
    in_specs=[
        _dpart_spec,
        pl.BlockSpec((NCORES, ROW_BLK, 64), lambda i: (0, i, 0)),
        pl.BlockSpec((1, 128), lambda i: (0, 0)),
        pl.BlockSpec((128, 64), lambda i: (0, 0)),
    ],
    out_specs=pl.BlockSpec((NCORES, ROW_BLK, 32), lambda i: (0, i, 0)),
    out_shape=jax.ShapeDtypeStruct((NCORES, N, 32), jnp.float32),
)

_tc3 = pl.pallas_call(
    _tc3_body,
    grid=(_N_BLKS,),
    in_specs=[
        _dpart_spec,
        pl.BlockSpec((NCORES, ROW_BLK, 32), lambda i: (0, i, 0)),
        pl.BlockSpec((1, 64), lambda i: (0, 0)),
    ],
    out_specs=pl.BlockSpec((ROW_BLK, 64), lambda i: (i, 0)),
    out_shape=jax.ShapeDtypeStruct((N, 64), jnp.float32),
)


@jax.jit
def kernel(x, edge_index, W1, b1, W2, b2):
    src = edge_index[0]
    dst = edge_index[1]
    dst_deg = dst.reshape(NCORES * NSUB, DEG_CHUNKS, CHUNK)
    src_agg = src.reshape(NSUB, AGG_CHUNKS, CHUNK)
    dst_agg = dst.reshape(NSUB, AGG_CHUNKS, CHUNK)
    ones_col = jnp.broadcast_to(
        (jnp.arange(16, dtype=jnp.int32) == 0).astype(jnp.float32), (N, 16))

    dpart = _deg_kernel(dst_deg, ones_col)
    hs1 = _tc1(dpart, x, W1)
    acc1 = _agg64(hs1, src_agg, dst_agg)
    hs2 = _tc2(dpart, acc1, b1.reshape(1, 128), W2)
    acc2 = _agg32(hs2, src_agg, dst_agg)
    return _tc3(dpart, acc2, b2.reshape(1, 64))


# speedup vs baseline: 34.2308x; 1.4080x over previous
import functools

import jax
import jax.numpy as jnp
from jax import lax
from jax.experimental import pallas as pl
from jax.experimental.pallas import tpu as pltpu
from jax.experimental.pallas import tpu_sc as plsc

N = 10000
E = 320000
NCORES = 2
NSUB = 16
STRIPE = 632
CHUNK = 100

DEG_EDGES_PER_W = E // (NCORES * NSUB)
DEG_CHUNKS = DEG_EDGES_PER_W // CHUNK
AGG_EDGES_PER_T = E // NSUB
AGG_CHUNKS = AGG_EDGES_PER_T // CHUNK

_MESH = plsc.VectorSubcoreMesh(core_axis_name="c", subcore_axis_name="s")
_SC_PARAMS = pltpu.CompilerParams(use_tc_tiling_on_sc=False)


def _deg_body(dst_hbm, ones_hbm, out_hbm, idx_v, ones_v, acc_sh, *sems):
    c = lax.axis_index("c")
    s = lax.axis_index("s")
    wid = s * NCORES + c
    base = pl.multiple_of(jnp.minimum(s * STRIPE, N - STRIPE), 8)
    pltpu.sync_copy(ones_hbm.at[pl.ds(base, STRIPE)],
                    acc_sh.at[pl.ds(base, STRIPE)])
    pltpu.sync_copy(dst_hbm.at[wid], idx_v)
    pltpu.sync_copy(ones_hbm.at[pl.ds(0, CHUNK)], ones_v)
    plsc.subcore_barrier()

    def body(j4, carry):
        for b in range(4):
            j = 4 * j4 + b

            @pl.when(j >= 4)
            def _():
                pltpu.make_async_copy(
                    ones_v, acc_sh.at[pl.ds(0, CHUNK)], sems[b]).wait()

            pltpu.async_copy(ones_v, acc_sh.at[idx_v.at[j]], sems[b], add=True)
        return carry

    lax.fori_loop(0, DEG_CHUNKS // 4, body, 0)
    for b in range(4):
        pltpu.make_async_copy(ones_v, acc_sh.at[pl.ds(0, CHUNK)],
                              sems[b]).wait()
    plsc.subcore_barrier()
    pltpu.sync_copy(acc_sh.at[pl.ds(base, STRIPE)],
                    out_hbm.at[c, pl.ds(base, STRIPE)])


_deg_kernel = pl.kernel(
    _deg_body,
    out_type=jax.ShapeDtypeStruct((NCORES, N, 16), jnp.float32),
    mesh=_MESH,
    compiler_params=_SC_PARAMS,
    scratch_types=[
        pltpu.VMEM((DEG_CHUNKS, CHUNK), jnp.int32),
        pltpu.VMEM((CHUNK, 16), jnp.float32),
        pltpu.VMEM_SHARED((N, 16), jnp.float32),
    ] + [pltpu.SemaphoreType.DMA] * 4,
)


def _agg_body(hs_hbm, src_hbm, dst_hbm, out_hbm,
              src_v, dst_v, rows, acc_sh, *sems, dh, nbuf, la):
    semg = sems[:nbuf]
    sems_ = sems[nbuf:]
    c = lax.axis_index("c")
    s = lax.axis_index("s")
    base = pl.multiple_of(jnp.minimum(s * STRIPE, N - STRIPE), 8)
    pltpu.sync_copy(hs_hbm.at[c, pl.ds(base, STRIPE)],
                    acc_sh.at[pl.ds(base, STRIPE)])
    pltpu.sync_copy(src_hbm.at[s], src_v)
    pltpu.sync_copy(dst_hbm.at[s], dst_v)
    plsc.subcore_barrier()

    def gather(j, b):
        pltpu.async_copy(hs_hbm.at[c].at[src_v.at[j]], rows.at[b], semg[b])

    def wait_gather(b):
        pltpu.make_async_copy(hs_hbm.at[c, pl.ds(0, CHUNK)], rows.at[b],
                              semg[b]).wait()

    def scatter(j, b):
        pltpu.async_copy(rows.at[b], acc_sh.at[dst_v.at[j]], sems_[b],
                         add=True)

    def wait_scatter(b):
        pltpu.make_async_copy(rows.at[b], acc_sh.at[pl.ds(0, CHUNK)],
                              sems_[b]).wait()

    for j in range(la):
        gather(j, j % nbuf)

    def body(jn, carry):
        for b in range(nbuf):
            j = nbuf * jn + b
            jg = j + la
            sg = (b + la) % nbuf

            @pl.when(jg < AGG_CHUNKS)
            def _():
                @pl.when(jg >= nbuf)
                def _():
                    wait_scatter(sg)

                gather(jg, sg)

            wait_gather(b)
            scatter(j, b)
        return carry

    lax.fori_loop(0, AGG_CHUNKS // nbuf, body, 0)
    for b in range(nbuf):
        wait_scatter(b)
    plsc.subcore_barrier()
    pltpu.sync_copy(acc_sh.at[pl.ds(base, STRIPE)],
                    out_hbm.at[c, pl.ds(base, STRIPE)])


def _make_agg_kernel(dh, nbuf, la):
    return pl.kernel(
        functools.partial(_agg_body, dh=dh, nbuf=nbuf, la=la),
        out_type=jax.ShapeDtypeStruct((NCORES, N, dh), jnp.float32),
        mesh=_MESH,
        compiler_params=_SC_PARAMS,
        scratch_types=[
            pltpu.VMEM((AGG_CHUNKS, CHUNK), jnp.int32),
            pltpu.VMEM((AGG_CHUNKS, CHUNK), jnp.int32),
            pltpu.VMEM((nbuf, CHUNK, dh), jnp.float32),
            pltpu.VMEM_SHARED((N, dh), jnp.float32),
        ] + [pltpu.SemaphoreType.DMA] * (2 * nbuf),
    )


_agg64 = _make_agg_kernel(64, 4, 2)
_agg32 = _make_agg_kernel(32, 8, 4)


ROW_BLK = 1000


def _dinv_block(dpart_ref):
    deg = dpart_ref[0, :, 0:1] + dpart_ref[1, :, 0:1] - 1.0
    return lax.rsqrt(deg)


def _tc1_body(dpart_ref, x_ref, w1_ref, out_ref):
    dinv = _dinv_block(dpart_ref)
    h = jnp.dot(x_ref[...], w1_ref[...], preferred_element_type=jnp.float32)
    hs = h * dinv
    out_ref[0, :, :] = hs[:, :64]
    out_ref[1, :, :] = hs[:, 64:]


def _tc2_body(dpart_ref, acc_ref, b1_ref, w2_ref, out_ref):
    dinv = _dinv_block(dpart_ref)
    z0 = jnp.maximum(acc_ref[0, :, :] * dinv + b1_ref[0, :64], 0.0)
    z1 = jnp.maximum(acc_ref[1, :, :] * dinv + b1_ref[0, 64:], 0.0)
    h = (jnp.dot(z0, w2_ref[:64, :], preferred_element_type=jnp.float32)
         + jnp.dot(z1, w2_ref[64:, :], preferred_element_type=jnp.float32))
    hs = h * dinv
    out_ref[0, :, :] = hs[:, :32]
    out_ref[1, :, :] = hs[:, 32:]


def _tc3_body(dpart_ref, acc_ref, b2_ref, out_ref):
    dinv = _dinv_block(dpart_ref)
    lo = acc_ref[0, :, :] * dinv + b2_ref[0, :32]
    hi = acc_ref[1, :, :] * dinv + b2_ref[0, 32:]
    out_ref[...] = jnp.concatenate([lo, hi], axis=1)


_N_BLKS = N // ROW_BLK

_dpart_spec = pl.BlockSpec((NCORES, ROW_BLK, 16), lambda i: (0, i, 0))

_tc1 = pl.pallas_call(
    _tc1_body,
    grid=(_N_BLKS,),
    in_specs=[
        _dpart_spec,
        pl.BlockSpec((ROW_BLK, 128), lambda i: (i, 0)),
        pl.BlockSpec((128, 128), lambda i: (0, 0)),
    ],
    out_specs=pl.BlockSpec((NCORES, ROW_BLK, 64), lambda i: (0, i, 0)),
    out_shape=jax.ShapeDtypeStruct((NCORES, N, 64), jnp.float32),
)

_tc2 = pl.pallas_call(
    _tc2_body,
    grid=(_N_BLKS,),
    in_specs=[
        _dpart_spec,
        pl.BlockSpec((NCORES, ROW_BLK, 64), lambda i: (0, i, 0)),
        pl.BlockSpec((1, 128), lambda i: (0, 0)),
        pl.BlockSpec((128, 64), lambda i: (0, 0)),
    ],
    out_specs=pl.BlockSpec((NCORES, ROW_BLK, 32), lambda i: (0, i, 0)),
    out_shape=jax.ShapeDtypeStruct((NCORES, N, 32), jnp.float32),
)

_tc3 = pl.pallas_call(
    _tc3_body,
    grid=(_N_BLKS,),
    in_specs=[
        _dpart_spec,
        pl.BlockSpec((NCORES, ROW_BLK, 32), lambda i: (0, i, 0)),
        pl.BlockSpec((1, 64), lambda i: (0, 0)),
    ],
    out_specs=pl.BlockSpec((ROW_BLK, 64), lambda i: (i, 0)),
    out_shape=jax.ShapeDtypeStruct((N, 64), jnp.float32),
)


@jax.jit
def kernel(x, edge_index, W1, b1, W2, b2):
    src = edge_index[0]
    dst = edge_index[1]
    dst_deg = dst.reshape(NCORES * NSUB, DEG_CHUNKS, CHUNK)
    src_agg = src.reshape(NSUB, AGG_CHUNKS, CHUNK)
    dst_agg = dst.reshape(NSUB, AGG_CHUNKS, CHUNK)
    ones_col = jnp.broadcast_to(
        (jnp.arange(16, dtype=jnp.int32) == 0).astype(jnp.float32), (N, 16))

    dpart = _deg_kernel(dst_deg, ones_col)
    hs1 = _tc1(dpart, x, W1)
    acc1 = _agg64(hs1, src_agg, dst_agg)
    hs2 = _tc2(dpart, acc1, b1.reshape(1, 128), W2)
    acc2 = _agg32(hs2, src_agg, dst_agg)
    return _tc3(dpart, acc2, b2.reshape(1, 64))


# speedup vs baseline: 36.0434x; 1.0530x over previous
import functools

import jax
import jax.numpy as jnp
from jax import lax
from jax.experimental import pallas as pl
from jax.experimental.pallas import tpu as pltpu
from jax.experimental.pallas import tpu_sc as plsc

N = 10000
E = 320000
NCORES = 2
NSUB = 16
STRIPE = 632
CHUNK = 100

DEG_EDGES_PER_W = E // (NCORES * NSUB)
DEG_CHUNKS = DEG_EDGES_PER_W // CHUNK
AGG_EDGES_PER_T = E // NSUB
AGG_CHUNKS = AGG_EDGES_PER_T // CHUNK

_MESH = plsc.VectorSubcoreMesh(core_axis_name="c", subcore_axis_name="s")
_SC_PARAMS = pltpu.CompilerParams(use_tc_tiling_on_sc=False)


def _deg_body(dst_hbm, ones_hbm, out_hbm, idx_v, ones_v, acc_sh, *sems):
    c = lax.axis_index("c")
    s = lax.axis_index("s")
    wid = s * NCORES + c
    base = pl.multiple_of(jnp.minimum(s * STRIPE, N - STRIPE), 8)
    pltpu.sync_copy(ones_hbm.at[pl.ds(base, STRIPE)],
                    acc_sh.at[pl.ds(base, STRIPE)])
    pltpu.sync_copy(dst_hbm.at[wid], idx_v)
    pltpu.sync_copy(ones_hbm.at[pl.ds(0, CHUNK)], ones_v)
    plsc.subcore_barrier()

    def body(j4, carry):
        for b in range(4):
            j = 4 * j4 + b

            @pl.when(j >= 4)
            def _():
                pltpu.make_async_copy(
                    ones_v, acc_sh.at[pl.ds(0, CHUNK)], sems[b]).wait()

            pltpu.async_copy(ones_v, acc_sh.at[idx_v.at[j]], sems[b], add=True)
        return carry

    lax.fori_loop(0, DEG_CHUNKS // 4, body, 0)
    for b in range(4):
        pltpu.make_async_copy(ones_v, acc_sh.at[pl.ds(0, CHUNK)],
                              sems[b]).wait()
    plsc.subcore_barrier()
    pltpu.sync_copy(acc_sh.at[pl.ds(base, STRIPE)],
                    out_hbm.at[c, pl.ds(base, STRIPE)])


_deg_kernel = pl.kernel(
    _deg_body,
    out_type=jax.ShapeDtypeStruct((NCORES, N, 16), jnp.float32),
    mesh=_MESH,
    compiler_params=_SC_PARAMS,
    scratch_types=[
        pltpu.VMEM((DEG_CHUNKS, CHUNK), jnp.int32),
        pltpu.VMEM((CHUNK, 16), jnp.float32),
        pltpu.VMEM_SHARED((N, 16), jnp.float32),
    ] + [pltpu.SemaphoreType.DMA] * 4,
)


def _agg_body(hs_hbm, src_hbm, dst_hbm, out_hbm,
              src_v, dst_v, rows, acc_sh, *sems, dh, nbuf, la):
    _agg_core(hs_hbm, src_hbm, dst_hbm, out_hbm,
              src_v, dst_v, rows, acc_sh, sems, dh, nbuf, la)


def _agg_core(hs_hbm, src_hbm, dst_hbm, out_hbm,
              src_v, dst_v, rows, acc_sh, sems, dh, nbuf, la,
              epilogue=None):
    semg = sems[:nbuf]
    sems_ = sems[nbuf:]
    c = lax.axis_index("c")
    s = lax.axis_index("s")
    base = pl.multiple_of(jnp.minimum(s * STRIPE, N - STRIPE), 8)
    pltpu.sync_copy(hs_hbm.at[c, pl.ds(base, STRIPE)],
                    acc_sh.at[pl.ds(base, STRIPE)])
    pltpu.sync_copy(src_hbm.at[s], src_v)
    pltpu.sync_copy(dst_hbm.at[s], dst_v)
    plsc.subcore_barrier()

    def gather(j, b):
        pltpu.async_copy(hs_hbm.at[c].at[src_v.at[j]], rows.at[b], semg[b])

    def wait_gather(b):
        pltpu.make_async_copy(hs_hbm.at[c, pl.ds(0, CHUNK)], rows.at[b],
                              semg[b]).wait()

    def scatter(j, b):
        pltpu.async_copy(rows.at[b], acc_sh.at[dst_v.at[j]], sems_[b],
                         add=True)

    def wait_scatter(b):
        pltpu.make_async_copy(rows.at[b], acc_sh.at[pl.ds(0, CHUNK)],
                              sems_[b]).wait()

    for j in range(la):
        gather(j, j % nbuf)

    def body(jn, carry):
        for b in range(nbuf):
            j = nbuf * jn + b
            jg = j + la
            sg = (b + la) % nbuf

            @pl.when(jg < AGG_CHUNKS)
            def _():
                @pl.when(jg >= nbuf)
                def _():
                    wait_scatter(sg)

                gather(jg, sg)

            wait_gather(b)
            scatter(j, b)
        return carry

    lax.fori_loop(0, AGG_CHUNKS // nbuf, body, 0)
    for b in range(nbuf):
        wait_scatter(b)
    plsc.subcore_barrier()
    if epilogue is None:
        pltpu.sync_copy(acc_sh.at[pl.ds(base, STRIPE)],
                        out_hbm.at[c, pl.ds(base, STRIPE)])
    else:
        epilogue(c, base)


def _agg_fin_body(hs_hbm, src_hbm, dst_hbm, dinv_hbm, b2_hbm, out_hbm,
                  src_v, dst_v, rows, zbuf, dbuf, b2v, acc_sh, *sems,
                  dh, nbuf, la):
    def epilogue(c, base):
        pltpu.sync_copy(acc_sh.at[pl.ds(base, STRIPE)], zbuf)
        pltpu.sync_copy(dinv_hbm.at[pl.ds(base, STRIPE)], dbuf)
        pltpu.sync_copy(b2_hbm.at[c], b2v)
        b2lo = b2v[0, :]
        b2hi = b2v[1, :]

        def row(r, carry):
            dv = dbuf[r, :]
            zbuf[r, 0:16] = zbuf[r, 0:16] * dv + b2lo
            zbuf[r, 16:32] = zbuf[r, 16:32] * dv + b2hi
            return carry

        lax.fori_loop(0, STRIPE, row, 0)
        pltpu.sync_copy(zbuf,
                        out_hbm.at[pl.ds(base, STRIPE), pl.ds(32 * c, 32)])

    _agg_core(hs_hbm, src_hbm, dst_hbm, out_hbm,
              src_v, dst_v, rows, acc_sh, sems, dh, nbuf, la,
              epilogue=epilogue)


_agg_fin = pl.kernel(
    functools.partial(_agg_fin_body, dh=32, nbuf=8, la=4),
    out_type=jax.ShapeDtypeStruct((N, 64), jnp.float32),
    mesh=_MESH,
    compiler_params=_SC_PARAMS,
    scratch_types=[
        pltpu.VMEM((AGG_CHUNKS, CHUNK), jnp.int32),
        pltpu.VMEM((AGG_CHUNKS, CHUNK), jnp.int32),
        pltpu.VMEM((8, CHUNK, 32), jnp.float32),
        pltpu.VMEM((STRIPE, 32), jnp.float32),
        pltpu.VMEM((STRIPE, 16), jnp.float32),
        pltpu.VMEM((2, 16), jnp.float32),
        pltpu.VMEM_SHARED((N, 32), jnp.float32),
    ] + [pltpu.SemaphoreType.DMA] * 16,
)


def _make_agg_kernel(dh, nbuf, la):
    return pl.kernel(
        functools.partial(_agg_body, dh=dh, nbuf=nbuf, la=la),
        out_type=jax.ShapeDtypeStruct((NCORES, N, dh), jnp.float32),
        mesh=_MESH,
        compiler_params=_SC_PARAMS,
        scratch_types=[
            pltpu.VMEM((AGG_CHUNKS, CHUNK), jnp.int32),
            pltpu.VMEM((AGG_CHUNKS, CHUNK), jnp.int32),
            pltpu.VMEM((nbuf, CHUNK, dh), jnp.float32),
            pltpu.VMEM_SHARED((N, dh), jnp.float32),
        ] + [pltpu.SemaphoreType.DMA] * (2 * nbuf),
    )


_agg64 = _make_agg_kernel(64, 4, 2)


ROW_BLK = 1000


def _dinv_block(dpart_ref):
    deg = dpart_ref[0, :, 0:1] + dpart_ref[1, :, 0:1] - 1.0
    return lax.rsqrt(deg)


def _tc1_body(dpart_ref, x_ref, w1_ref, out_ref, dinv_ref):
    dinv = _dinv_block(dpart_ref)
    h = jnp.dot(x_ref[...], w1_ref[...], preferred_element_type=jnp.float32)
    hs = h * dinv
    out_ref[0, :, :] = hs[:, :64]
    out_ref[1, :, :] = hs[:, 64:]
    dinv_ref[...] = jnp.broadcast_to(dinv, (ROW_BLK, 16))


def _tc2_body(dinv16_ref, acc_ref, b1_ref, w2_ref, out_ref):
    dinv = dinv16_ref[:, 0:1]
    z0 = jnp.maximum(acc_ref[0, :, :] * dinv + b1_ref[0, :64], 0.0)
    z1 = jnp.maximum(acc_ref[1, :, :] * dinv + b1_ref[0, 64:], 0.0)
    h = (jnp.dot(z0, w2_ref[:64, :], preferred_element_type=jnp.float32)
         + jnp.dot(z1, w2_ref[64:, :], preferred_element_type=jnp.float32))
    hs = h * dinv
    out_ref[0, :, :] = hs[:, :32]
    out_ref[1, :, :] = hs[:, 32:]


_N_BLKS = N // ROW_BLK

_dpart_spec = pl.BlockSpec((NCORES, ROW_BLK, 16), lambda i: (0, i, 0))

_tc1 = pl.pallas_call(
    _tc1_body,
    grid=(_N_BLKS,),
    in_specs=[
        _dpart_spec,
        pl.BlockSpec((ROW_BLK, 128), lambda i: (i, 0)),
        pl.BlockSpec((128, 128), lambda i: (0, 0)),
    ],
    out_specs=[pl.BlockSpec((NCORES, ROW_BLK, 64), lambda i: (0, i, 0)),
               pl.BlockSpec((ROW_BLK, 16), lambda i: (i, 0))],
    out_shape=[jax.ShapeDtypeStruct((NCORES, N, 64), jnp.float32),
               jax.ShapeDtypeStruct((N, 16), jnp.float32)],
)

_tc2 = pl.pallas_call(
    _tc2_body,
    grid=(_N_BLKS,),
    in_specs=[
        pl.BlockSpec((ROW_BLK, 16), lambda i: (i, 0)),
        pl.BlockSpec((NCORES, ROW_BLK, 64), lambda i: (0, i, 0)),
        pl.BlockSpec((1, 128), lambda i: (0, 0)),
        pl.BlockSpec((128, 64), lambda i: (0, 0)),
    ],
    out_specs=pl.BlockSpec((NCORES, ROW_BLK, 32), lambda i: (0, i, 0)),
    out_shape=jax.ShapeDtypeStruct((NCORES, N, 32), jnp.float32),
)

@jax.jit
def kernel(x, edge_index, W1, b1, W2, b2):
    src = edge_index[0]
    dst = edge_index[1]
    dst_deg = dst.reshape(NCORES * NSUB, DEG_CHUNKS, CHUNK)
    src_agg = src.reshape(NSUB, AGG_CHUNKS, CHUNK)
    dst_agg = dst.reshape(NSUB, AGG_CHUNKS, CHUNK)
    ones_col = jnp.broadcast_to(
        (jnp.arange(16, dtype=jnp.int32) == 0).astype(jnp.float32), (N, 16))

    dpart = _deg_kernel(dst_deg, ones_col)
    hs1, dinv16 = _tc1(dpart, x, W1)
    acc1 = _agg64(hs1, src_agg, dst_agg)
    hs2 = _tc2(dinv16, acc1, b1.reshape(1, 128), W2)
    return _agg_fin(hs2, src_agg, dst_agg, dinv16,
                    b2.reshape(NCORES, 2, 16))


# speedup vs baseline: 39.7370x; 1.1025x over previous
import functools

import jax
import jax.numpy as jnp
from jax import lax
from jax.experimental import pallas as pl
from jax.experimental.pallas import tpu as pltpu
from jax.experimental.pallas import tpu_sc as plsc

N = 10000
E = 320000
NCORES = 2
NSUB = 16
STRIPE = 632
CHUNK = 125

DEG_EDGES_PER_W = E // (NCORES * NSUB)
DEG_CHUNKS = DEG_EDGES_PER_W // CHUNK
AGG_EDGES_PER_T = E // NSUB
AGG_CHUNKS = AGG_EDGES_PER_T // CHUNK

_MESH = plsc.VectorSubcoreMesh(core_axis_name="c", subcore_axis_name="s")
_SC_PARAMS = pltpu.CompilerParams(use_tc_tiling_on_sc=False)


def _deg_body(dst_hbm, zeros_hbm, ones_hbm, out_hbm, idx_v, ones_v, acc_sh,
              *sems):
    c = lax.axis_index("c")
    s = lax.axis_index("s")
    wid = s * NCORES + c
    base = pl.multiple_of(jnp.minimum(s * STRIPE, N - STRIPE), 8)
    pltpu.sync_copy(zeros_hbm.at[:, pl.ds(0, 16)],
                    acc_sh.at[pl.ds(base, STRIPE)])
    pltpu.sync_copy(dst_hbm.at[wid], idx_v)
    pltpu.sync_copy(ones_hbm, ones_v)
    plsc.subcore_barrier()

    def body(j4, carry):
        for b in range(4):
            j = 4 * j4 + b

            @pl.when(j >= 4)
            def _():
                pltpu.make_async_copy(
                    ones_v, acc_sh.at[pl.ds(0, CHUNK)], sems[b]).wait()

            pltpu.async_copy(ones_v, acc_sh.at[idx_v.at[j]], sems[b], add=True)
        return carry

    lax.fori_loop(0, DEG_CHUNKS // 4, body, 0)
    for b in range(4):
        pltpu.make_async_copy(ones_v, acc_sh.at[pl.ds(0, CHUNK)],
                              sems[b]).wait()
    plsc.subcore_barrier()
    pltpu.sync_copy(acc_sh.at[pl.ds(base, STRIPE)],
                    out_hbm.at[c, pl.ds(base, STRIPE)])


_deg_kernel = pl.kernel(
    _deg_body,
    out_type=jax.ShapeDtypeStruct((NCORES, N, 16), jnp.float32),
    mesh=_MESH,
    compiler_params=_SC_PARAMS,
    scratch_types=[
        pltpu.VMEM((DEG_CHUNKS, CHUNK), jnp.int32),
        pltpu.VMEM((CHUNK, 16), jnp.float32),
        pltpu.VMEM_SHARED((N, 16), jnp.float32),
    ] + [pltpu.SemaphoreType.DMA] * 4,
)


def _agg_core(zeros_hbm, tab_hbm, src_hbm, dst_hbm, out_hbm,
              src_v, dst_v, rows, acc_sh, sems, dh, nbuf, la, chunks,
              epilogue=None):
    semg = sems[:nbuf]
    sems_ = sems[nbuf:]
    c = lax.axis_index("c")
    s = lax.axis_index("s")
    base = pl.multiple_of(jnp.minimum(s * STRIPE, N - STRIPE), 8)
    pltpu.sync_copy(zeros_hbm.at[:, pl.ds(0, dh)],
                    acc_sh.at[pl.ds(base, STRIPE)])
    pltpu.sync_copy(src_hbm.at[c, s], src_v)
    pltpu.sync_copy(dst_hbm.at[s], dst_v)
    plsc.subcore_barrier()

    def gather(j, b):
        pltpu.async_copy(tab_hbm.at[src_v.at[j]], rows.at[b], semg[b])

    def wait_gather(b):
        pltpu.make_async_copy(tab_hbm.at[pl.ds(0, CHUNK)], rows.at[b],
                              semg[b]).wait()

    def scatter(j, b):
        pltpu.async_copy(rows.at[b], acc_sh.at[dst_v.at[j]], sems_[b],
                         add=True)

    def wait_scatter(b):
        pltpu.make_async_copy(rows.at[b], acc_sh.at[pl.ds(0, CHUNK)],
                              sems_[b]).wait()

    for j in range(la):
        gather(j, j % nbuf)

    def body(jn, carry):
        for b in range(nbuf):
            j = nbuf * jn + b
            jg = j + la
            sg = (b + la) % nbuf

            @pl.when(jg < chunks)
            def _():
                @pl.when(jg >= nbuf)
                def _():
                    wait_scatter(sg)

                gather(jg, sg)

            wait_gather(b)
            scatter(j, b)
        return carry

    lax.fori_loop(0, chunks // nbuf, body, 0)
    for b in range(nbuf):
        wait_scatter(b)
    plsc.subcore_barrier()
    if epilogue is None:
        pltpu.sync_copy(acc_sh.at[pl.ds(base, STRIPE)],
                        out_hbm.at[pl.ds(base, STRIPE), pl.ds(dh * c, dh)])
    else:
        epilogue(c, base)


def _agg_body(zeros_hbm, tab_hbm, src_hbm, dst_hbm, out_hbm,
              src_v, dst_v, rows, acc_sh, *sems, dh, nbuf, la, chunks):
    _agg_core(zeros_hbm, tab_hbm, src_hbm, dst_hbm, out_hbm,
              src_v, dst_v, rows, acc_sh, sems, dh, nbuf, la, chunks)


def _agg_fin_body(zeros_hbm, tab_hbm, src_hbm, dst_hbm, dinv_hbm, b2_hbm,
                  out_hbm, src_v, dst_v, rows, zbuf, dbuf, b2v, acc_sh,
                  *sems, dh, nbuf, la, chunks):
    def epilogue(c, base):
        pltpu.sync_copy(acc_sh.at[pl.ds(base, STRIPE)], zbuf)
        pltpu.sync_copy(dinv_hbm.at[pl.ds(base, STRIPE), pl.ds(0, 16)], dbuf)
        pltpu.sync_copy(b2_hbm.at[c], b2v)
        b2lo = b2v[0, :]
        b2hi = b2v[1, :]

        def row(r, carry):
            dv = dbuf[r, :]
            zbuf[r, 0:16] = zbuf[r, 0:16] * dv + b2lo
            zbuf[r, 16:32] = zbuf[r, 16:32] * dv + b2hi
            return carry

        lax.fori_loop(0, STRIPE, row, 0)
        pltpu.sync_copy(zbuf,
                        out_hbm.at[pl.ds(base, STRIPE), pl.ds(32 * c, 32)])

    _agg_core(zeros_hbm, tab_hbm, src_hbm, dst_hbm, out_hbm,
              src_v, dst_v, rows, acc_sh, sems, dh, nbuf, la, chunks,
              epilogue=epilogue)


FIN_EDGES = E + N + 6000
FIN_CHUNKS = FIN_EDGES // (NSUB * CHUNK)


_agg64 = pl.kernel(
    functools.partial(_agg_body, dh=64, nbuf=5, la=2, chunks=AGG_CHUNKS),
    out_type=jax.ShapeDtypeStruct((N, 128), jnp.float32),
    mesh=_MESH,
    compiler_params=_SC_PARAMS,
    scratch_types=[
        pltpu.VMEM((AGG_CHUNKS, CHUNK), jnp.int32),
        pltpu.VMEM((AGG_CHUNKS, CHUNK), jnp.int32),
        pltpu.VMEM((5, CHUNK, 64), jnp.float32),
        pltpu.VMEM_SHARED((N, 64), jnp.float32),
    ] + [pltpu.SemaphoreType.DMA] * 10,
)

_agg_fin = pl.kernel(
    functools.partial(_agg_fin_body, dh=32, nbuf=8, la=4, chunks=FIN_CHUNKS),
    out_type=jax.ShapeDtypeStruct((N, 64), jnp.float32),
    mesh=_MESH,
    compiler_params=_SC_PARAMS,
    scratch_types=[
        pltpu.VMEM((FIN_CHUNKS, CHUNK), jnp.int32),
        pltpu.VMEM((FIN_CHUNKS, CHUNK), jnp.int32),
        pltpu.VMEM((8, CHUNK, 32), jnp.float32),
        pltpu.VMEM((STRIPE, 32), jnp.float32),
        pltpu.VMEM((STRIPE, 16), jnp.float32),
        pltpu.VMEM((2, 16), jnp.float32),
        pltpu.VMEM_SHARED((N, 32), jnp.float32),
    ] + [pltpu.SemaphoreType.DMA] * 16,
)


ROW_BLK = 1000


def _dinv_block(dpart_ref):
    deg = dpart_ref[0, :, 0:1] + dpart_ref[1, :, 0:1] + 1.0
    return lax.rsqrt(deg)


def _tc1_body(dpart_ref, x_ref, w1_ref, out_ref, dinv_ref):
    dinv = _dinv_block(dpart_ref)
    h = jnp.dot(x_ref[...], w1_ref[...], preferred_element_type=jnp.float32)
    out_ref[...] = h * dinv
    dinv_ref[...] = jnp.broadcast_to(dinv, (ROW_BLK, 128))


def _tc2_body(dinv_ref, acc_ref, hs1_ref, b1_ref, w2_ref, out_ref):
    z = jnp.maximum((acc_ref[...] + hs1_ref[...]) * dinv_ref[...]
                    + b1_ref[0, :], 0.0)
    h = jnp.dot(z, w2_ref[...], preferred_element_type=jnp.float32)
    hs = h * dinv_ref[:, 0:1]
    out_ref[...] = jnp.concatenate(
        [hs, jnp.zeros((ROW_BLK, 64), jnp.float32)], axis=1)


_N_BLKS = N // ROW_BLK

_dpart_spec = pl.BlockSpec((NCORES, ROW_BLK, 16), lambda i: (0, i, 0))

_tc1 = pl.pallas_call(
    _tc1_body,
    grid=(_N_BLKS,),
    in_specs=[
        _dpart_spec,
        pl.BlockSpec((ROW_BLK, 128), lambda i: (i, 0)),
        pl.BlockSpec((128, 128), lambda i: (0, 0)),
    ],
    out_specs=[pl.BlockSpec((ROW_BLK, 128), lambda i: (i, 0)),
               pl.BlockSpec((ROW_BLK, 128), lambda i: (i, 0))],
    out_shape=[jax.ShapeDtypeStruct((N, 128), jnp.float32),
               jax.ShapeDtypeStruct((N, 128), jnp.float32)],
)

_tc2 = pl.pallas_call(
    _tc2_body,
    grid=(_N_BLKS,),
    in_specs=[
        pl.BlockSpec((ROW_BLK, 128), lambda i: (i, 0)),
        pl.BlockSpec((ROW_BLK, 128), lambda i: (i, 0)),
        pl.BlockSpec((ROW_BLK, 128), lambda i: (i, 0)),
        pl.BlockSpec((1, 128), lambda i: (0, 0)),
        pl.BlockSpec((128, 64), lambda i: (0, 0)),
    ],
    out_specs=pl.BlockSpec((ROW_BLK, 128), lambda i: (i, 0)),
    out_shape=jax.ShapeDtypeStruct((N, 128), jnp.float32),
)


@jax.jit
def kernel(x, edge_index, W1, b1, W2, b2):
    src = edge_index[0]
    dst = edge_index[1]
    cvec = jnp.arange(NCORES, dtype=jnp.int32)[:, None]
    loop = jnp.arange(N, dtype=jnp.int32)
    dummy_n = FIN_EDGES - E - N
    dst_deg = dst.reshape(NCORES * NSUB, DEG_CHUNKS, CHUNK)
    dst_agg = dst.reshape(NSUB, AGG_CHUNKS, CHUNK)
    src2 = (2 * src[None, :] + cvec).reshape(NCORES, NSUB, AGG_CHUNKS, CHUNK)
    dummy = jnp.arange(dummy_n, dtype=jnp.int32)
    srcf = jnp.concatenate(
        [4 * src, 4 * loop, 4 * dummy + 2])[None, :] + cvec
    srcf = srcf.reshape(NCORES, NSUB, FIN_CHUNKS, CHUNK)
    dstf = jnp.concatenate([dst, loop, dummy]).reshape(
        NSUB, FIN_CHUNKS, CHUNK)
    ones_sm = jnp.broadcast_to(
        (jnp.arange(16, dtype=jnp.int32) == 0).astype(jnp.float32),
        (CHUNK, 16))
    zeros_sm = jnp.zeros((STRIPE, 128), jnp.float32)

    dpart = _deg_kernel(dst_deg, zeros_sm, ones_sm)
    hs1, dinv128 = _tc1(dpart, x, W1)
    acc1 = _agg64(zeros_sm, hs1.reshape(2 * N, 64), src2, dst_agg)
    hs2 = _tc2(dinv128, acc1, hs1, b1.reshape(1, 128), W2)
    return _agg_fin(zeros_sm, hs2.reshape(4 * N, 32), srcf, dstf,
                    dinv128, b2.reshape(NCORES, 2, 16))
